# Initial kernel scaffold; baseline (speedup 1.0000x reference)
import jax
import jax.numpy as jnp
from jax.experimental import pallas as pl


def kernel(coords, ref):
    raise NotImplementedError("write your pallas kernel here")


# speedup vs baseline: 4.4314x; 4.4314x over previous
import functools

import jax
import jax.numpy as jnp
from jax import lax
from jax.experimental import pallas as pl
from jax.experimental.pallas import tpu as pltpu
from jax.experimental.pallas import tpu_sc as plsc

B = 16
N = 4096
MAX_INCLUDED = 512
SQ_CUT = 1.0
L = 16
CHUNKS = N // L
OUT_WORDS = MAX_INCLUDED * 3


def _sc_body(coords_hbm, ref_hbm, out_hbm, cbuf, rbuf, obuf):
    c = lax.axis_index("c")
    s = lax.axis_index("s")

    @pl.when(s < B // 2)
    def _():
        row = c * (B // 2) + s

        pltpu.sync_copy(coords_hbm.at[row], cbuf)
        pltpu.sync_copy(ref_hbm.at[row], rbuf)

        zeros_f = jnp.zeros((L,), jnp.float32)
        zeros_i = jnp.zeros((L,), jnp.int32)
        lane = lax.iota(jnp.int32, L)
        lane3 = lane * 3

        def zbody(j, carry):
            obuf[pl.ds(j * L, L)] = zeros_f
            return carry
        lax.fori_loop(0, OUT_WORDS // L, zbody, 0)

        rx = rbuf[pl.ds(0, L)]
        ry = rbuf[pl.ds(L, L)]
        rz = rbuf[pl.ds(2 * L, L)]

        def body(i, off):
            ix = lane3 + i * (3 * L)
            x = plsc.load_gather(cbuf, [ix])
            y = plsc.load_gather(cbuf, [ix + 1])
            z = plsc.load_gather(cbuf, [ix + 2])
            dx = x - rx
            dy = y - ry
            dz = z - rz
            d2 = dx * dx + dy * dy + dz * dz
            m = d2 <= SQ_CUT
            pos = off + plsc.cumsum(m.astype(jnp.int32)) - 1
            valid = m & (pos < MAX_INCLUDED)
            fidx = pos * 3
            plsc.store_scatter(obuf, [fidx], dx, mask=valid)
            plsc.store_scatter(obuf, [fidx + 1], dy, mask=valid)
            plsc.store_scatter(obuf, [fidx + 2], dz, mask=valid)
            return off + plsc.all_reduce_population_count(m)

        lax.fori_loop(0, CHUNKS, body, zeros_i)

        pltpu.sync_copy(obuf, out_hbm.at[row])


@jax.jit
def _run(coords_flat, ref_pad):
    mesh = plsc.VectorSubcoreMesh(core_axis_name="c", subcore_axis_name="s")
    k = functools.partial(
        pl.kernel,
        mesh=mesh,
        out_type=jax.ShapeDtypeStruct((B, OUT_WORDS), jnp.float32),
        compiler_params=pltpu.CompilerParams(needs_layout_passes=False),
        scratch_types=[
            pltpu.VMEM((N * 3,), jnp.float32),
            pltpu.VMEM((3 * L,), jnp.float32),
            pltpu.VMEM((OUT_WORDS,), jnp.float32),
        ],
    )(_sc_body)
    return k(coords_flat, ref_pad)


def kernel(coords, ref):
    coords_flat = coords.reshape(B, N * 3)
    ref_pad = jnp.broadcast_to(ref[:, :, None], (B, 3, L)).reshape(B, 3 * L)
    out = _run(coords_flat, ref_pad)
    return out.reshape(B, MAX_INCLUDED, 3)
